# Initial kernel scaffold; baseline (speedup 1.0000x reference)
import jax
import jax.numpy as jnp
from jax.experimental import pallas as pl


def kernel(x, w1, b1, w2, b2, w1t, b1t, w2t, b2t):
    raise NotImplementedError("write your pallas kernel here")


# speedup vs baseline: 1.0212x; 1.0212x over previous
import functools

import jax
import jax.numpy as jnp
from jax.experimental import pallas as pl
from jax.experimental.pallas import tpu as pltpu

_ROW_TILE = 2048
_CHUNK = 128


def _spatial_reduce_kernel(x_ref, s_ref, m_ref):
    rows = s_ref.shape[0]
    chunk = _CHUNK if rows % _CHUNK == 0 else rows
    for k in range(rows // chunk):
        sl = slice(k * chunk, (k + 1) * chunk)
        blk = x_ref[sl, :]
        s_ref[sl, :] = jnp.sum(blk, axis=1, keepdims=True)
        m_ref[sl, :] = jnp.max(blk, axis=1, keepdims=True)


def _gate_kernel(s_ref, m_ref, w1_ref, b1_ref, w2_ref, b2_ref,
                 w1t_ref, b1t_ref, w2t_ref, b2t_ref, out_ref, *,
                 inv_thw, inv_chw):
    f32 = jnp.float32
    c00 = (((0,), (0,)), ((), ()))
    S = s_ref[...]
    M = m_ref[...]
    w1 = w1_ref[...]
    b1 = b1_ref[...]

    gs = jax.lax.dot_general(S, w1, c00, preferred_element_type=f32)
    ga = jnp.sum(gs, axis=0, keepdims=True) * inv_thw
    m1 = jnp.max(M, axis=1, keepdims=True)
    gm = jax.lax.dot_general(m1, w1, c00, preferred_element_type=f32)

    ha = jnp.maximum(ga + b1, 0.0)
    hm = jnp.maximum(gm + b1, 0.0)
    w2 = w2_ref[...]
    o1 = (jnp.dot(ha, w2, preferred_element_type=f32)
          + jnp.dot(hm, w2, preferred_element_type=f32)
          + 2.0 * b2_ref[...])
    scale = jax.nn.sigmoid(o1)

    pa = jnp.dot(scale, S, preferred_element_type=f32) * inv_chw
    mt = M.T
    pm = jnp.max(mt * scale, axis=1, keepdims=True).T

    w1t = w1t_ref[...]
    b1t = b1t_ref[...]
    h2a = jnp.maximum(jnp.dot(pa, w1t, preferred_element_type=f32) + b1t, 0.0)
    h2m = jnp.maximum(jnp.dot(pm, w1t, preferred_element_type=f32) + b1t, 0.0)
    w2t = w2t_ref[...]
    o2 = (jnp.dot(h2a, w2t, preferred_element_type=f32)
          + jnp.dot(h2m, w2t, preferred_element_type=f32)
          + 2.0 * b2t_ref[...])
    out_ref[...] = jax.nn.sigmoid(o2)[None]


def kernel(x, w1, b1, w2, b2, w1t, b1t, w2t, b2t):
    B, C, T, H, W = x.shape
    HW = H * W
    N = B * C * T

    x2 = x.reshape(N, HW)
    row_tile = _ROW_TILE if N % _ROW_TILE == 0 else N
    grid1 = N // row_tile
    imap = lambda i: (i, 0)
    sums, maxes = pl.pallas_call(
        _spatial_reduce_kernel,
        out_shape=(jax.ShapeDtypeStruct((N, 1), jnp.float32),
                   jax.ShapeDtypeStruct((N, 1), jnp.float32)),
        grid=(grid1,),
        in_specs=[pl.BlockSpec((row_tile, HW), imap)],
        out_specs=(pl.BlockSpec((row_tile, 1), imap),
                   pl.BlockSpec((row_tile, 1), imap)),
        compiler_params=pltpu.CompilerParams(
            dimension_semantics=("arbitrary",)),
    )(x2)

    S = sums.reshape(B * C, T)
    M = maxes.reshape(B * C, T)

    gate = functools.partial(
        _gate_kernel,
        inv_thw=1.0 / float(T * H * W), inv_chw=1.0 / float(C * H * W))
    bmap2 = lambda b: (b, 0)
    bmap3 = lambda b: (b, 0, 0)
    zmap = lambda shape: (lambda b: tuple(0 for _ in shape))
    full = lambda shape: pl.BlockSpec(shape, zmap(shape))
    mc2 = pl.pallas_call(
        gate,
        out_shape=jax.ShapeDtypeStruct((B, 1, T), jnp.float32),
        grid=(B,),
        in_specs=[pl.BlockSpec((C, T), bmap2),
                  pl.BlockSpec((C, T), bmap2),
                  full(w1.shape), full(b1.shape), full(w2.shape),
                  full(b2.shape), full(w1t.T.shape), full((1, b1t.shape[0])),
                  full(w2t.T.shape), full((1, b2t.shape[0]))],
        out_specs=pl.BlockSpec((1, 1, T), bmap3),
        compiler_params=pltpu.CompilerParams(
            dimension_semantics=("arbitrary",)),
    )(S, M, w1, b1, w2, b2, w1t.T, b1t.T, w2t.T, b2t.T)
    return mc2.reshape(B, T)


# speedup vs baseline: 1.6681x; 1.6335x over previous
import functools

import jax
import jax.numpy as jnp
from jax.experimental import pallas as pl
from jax.experimental.pallas import tpu as pltpu

_ROW_TILE = 1024
_CHUNK = 128


def _spatial_reduce_kernel(x_ref, s_ref, m_ref):
    rows = s_ref.shape[0]
    chunk = _CHUNK if rows % _CHUNK == 0 else rows
    for k in range(rows // chunk):
        sl = slice(k * chunk, (k + 1) * chunk)
        blk = x_ref[sl, :, :]
        zs = jnp.sum(blk, axis=1)
        zm = jnp.max(blk, axis=1)
        s_ref[sl, :] = jnp.sum(zs, axis=1, keepdims=True)
        m_ref[sl, :] = jnp.max(zm, axis=1, keepdims=True)


def _gate_kernel(s_ref, m_ref, w1_ref, b1_ref, w2_ref, b2_ref,
                 w1t_ref, b1t_ref, w2t_ref, b2t_ref, out_ref, *,
                 inv_thw, inv_chw):
    f32 = jnp.float32
    c00 = (((0,), (0,)), ((), ()))
    S = s_ref[...]
    M = m_ref[...]
    w1 = w1_ref[...]
    b1 = b1_ref[...]

    gs = jax.lax.dot_general(S, w1, c00, preferred_element_type=f32)
    ga = jnp.sum(gs, axis=0, keepdims=True) * inv_thw
    m1 = jnp.max(M, axis=1, keepdims=True)
    gm = jax.lax.dot_general(m1, w1, c00, preferred_element_type=f32)

    ha = jnp.maximum(ga + b1, 0.0)
    hm = jnp.maximum(gm + b1, 0.0)
    w2 = w2_ref[...]
    o1 = (jnp.dot(ha, w2, preferred_element_type=f32)
          + jnp.dot(hm, w2, preferred_element_type=f32)
          + 2.0 * b2_ref[...])
    scale = jax.nn.sigmoid(o1)

    pa = jnp.dot(scale, S, preferred_element_type=f32) * inv_chw
    mt = M.T
    pm = jnp.max(mt * scale, axis=1, keepdims=True).T

    w1t = w1t_ref[...]
    b1t = b1t_ref[...]
    h2a = jnp.maximum(jnp.dot(pa, w1t, preferred_element_type=f32) + b1t, 0.0)
    h2m = jnp.maximum(jnp.dot(pm, w1t, preferred_element_type=f32) + b1t, 0.0)
    w2t = w2t_ref[...]
    o2 = (jnp.dot(h2a, w2t, preferred_element_type=f32)
          + jnp.dot(h2m, w2t, preferred_element_type=f32)
          + 2.0 * b2t_ref[...])
    out_ref[...] = jax.nn.sigmoid(o2)[None]


def kernel(x, w1, b1, w2, b2, w1t, b1t, w2t, b2t):
    B, C, T, H, W = x.shape
    HW = H * W
    N = B * C * T

    x3 = x.reshape(N, H, W)
    row_tile = _ROW_TILE if N % _ROW_TILE == 0 else N
    grid1 = N // row_tile
    imap = lambda i: (i, 0)
    sums, maxes = pl.pallas_call(
        _spatial_reduce_kernel,
        out_shape=(jax.ShapeDtypeStruct((N, 1), jnp.float32),
                   jax.ShapeDtypeStruct((N, 1), jnp.float32)),
        grid=(grid1,),
        in_specs=[pl.BlockSpec((row_tile, H, W), lambda i: (i, 0, 0))],
        out_specs=(pl.BlockSpec((row_tile, 1), imap),
                   pl.BlockSpec((row_tile, 1), imap)),
        compiler_params=pltpu.CompilerParams(
            dimension_semantics=("arbitrary",)),
    )(x3)

    S = sums.reshape(B * C, T)
    M = maxes.reshape(B * C, T)

    gate = functools.partial(
        _gate_kernel,
        inv_thw=1.0 / float(T * H * W), inv_chw=1.0 / float(C * H * W))
    bmap2 = lambda b: (b, 0)
    bmap3 = lambda b: (b, 0, 0)
    zmap = lambda shape: (lambda b: tuple(0 for _ in shape))
    full = lambda shape: pl.BlockSpec(shape, zmap(shape))
    mc2 = pl.pallas_call(
        gate,
        out_shape=jax.ShapeDtypeStruct((B, 1, T), jnp.float32),
        grid=(B,),
        in_specs=[pl.BlockSpec((C, T), bmap2),
                  pl.BlockSpec((C, T), bmap2),
                  full(w1.shape), full(b1.shape), full(w2.shape),
                  full(b2.shape), full(w1t.T.shape), full((1, b1t.shape[0])),
                  full(w2t.T.shape), full((1, b2t.shape[0]))],
        out_specs=pl.BlockSpec((1, 1, T), bmap3),
        compiler_params=pltpu.CompilerParams(
            dimension_semantics=("arbitrary",)),
    )(S, M, w1, b1, w2, b2, w1t.T, b1t.T, w2t.T, b2t.T)
    return mc2.reshape(B, T)


# speedup vs baseline: 22.3563x; 13.4021x over previous
import functools

import jax
import jax.numpy as jnp
from jax.experimental import pallas as pl
from jax.experimental.pallas import tpu as pltpu

_SEG = 16


def _fused_kernel(x_ref, w1_ref, b1_ref, w2_ref, b2_ref,
                  w1t_ref, b1t_ref, w2t_ref, b2t_ref, out_ref, *,
                  inv_thw, inv_chw):
    f32 = jnp.float32
    hw = x_ref.shape[1]
    seg = _SEG if hw % _SEG == 0 else hw
    s_acc = None
    m_acc = None
    for k in range(hw // seg):
        blk = x_ref[0, k * seg:(k + 1) * seg]
        ps = jnp.sum(blk, axis=0)
        pm = jnp.max(blk, axis=0)
        s_acc = ps if s_acc is None else s_acc + ps
        m_acc = pm if m_acc is None else jnp.maximum(m_acc, pm)
    S, M = s_acc, m_acc

    a1 = jnp.sum(S, axis=0, keepdims=True) * inv_thw
    m1 = jnp.max(M, axis=0, keepdims=True)
    w1 = w1_ref[...]
    b1 = b1_ref[...]
    ha = jnp.maximum(jnp.dot(a1, w1, preferred_element_type=f32) + b1, 0.0)
    hm = jnp.maximum(jnp.dot(m1, w1, preferred_element_type=f32) + b1, 0.0)
    w2 = w2_ref[...]
    o1 = (jnp.dot(ha, w2, preferred_element_type=f32)
          + jnp.dot(hm, w2, preferred_element_type=f32)
          + 2.0 * b2_ref[...])
    scale = jax.nn.sigmoid(o1)

    pa = jnp.sum(S * scale, axis=1, keepdims=True).T * inv_chw
    pm2 = jnp.max(M * scale, axis=1, keepdims=True).T
    w1t = w1t_ref[...]
    b1t = b1t_ref[...]
    h2a = jnp.maximum(jnp.dot(pa, w1t, preferred_element_type=f32) + b1t, 0.0)
    h2m = jnp.maximum(jnp.dot(pm2, w1t, preferred_element_type=f32) + b1t, 0.0)
    w2t = w2t_ref[...]
    o2 = (jnp.dot(h2a, w2t, preferred_element_type=f32)
          + jnp.dot(h2m, w2t, preferred_element_type=f32)
          + 2.0 * b2t_ref[...])
    out_ref[...] = jax.nn.sigmoid(o2)[None]


def kernel(x, w1, b1, w2, b2, w1t, b1t, w2t, b2t):
    B, C, T, H, W = x.shape
    HW = H * W

    xt = jnp.transpose(x, (0, 3, 4, 2, 1)).reshape(B, HW, T, C)

    fused = functools.partial(
        _fused_kernel,
        inv_thw=1.0 / float(T * H * W), inv_chw=1.0 / float(C * H * W))
    zmap = lambda shape: (lambda b: tuple(0 for _ in shape))
    full = lambda a: pl.BlockSpec(a.shape, zmap(a.shape))
    w1t_r, b1t_r, w2t_r, b2t_r = w1t.T, b1t.T, w2t.T, b2t.T
    mc2 = pl.pallas_call(
        fused,
        out_shape=jax.ShapeDtypeStruct((B, 1, T), jnp.float32),
        grid=(B,),
        in_specs=[pl.BlockSpec((1, HW, T, C), lambda b: (b, 0, 0, 0)),
                  full(w1), full(b1), full(w2), full(b2),
                  full(w1t_r), full(b1t_r), full(w2t_r), full(b2t_r)],
        out_specs=pl.BlockSpec((1, 1, T), lambda b: (b, 0, 0)),
        compiler_params=pltpu.CompilerParams(
            dimension_semantics=("arbitrary",)),
    )(xt, w1, b1, w2, b2, w1t_r, b1t_r, w2t_r, b2t_r)
    return mc2.reshape(B, T)


# speedup vs baseline: 23.5910x; 1.0552x over previous
import functools

import jax
import jax.numpy as jnp
from jax.experimental import pallas as pl
from jax.experimental.pallas import tpu as pltpu

_SEG = 16


def _fused_kernel(x_ref, w1_ref, b1_ref, w2_ref, b2_ref,
                  w1t_ref, b1t_ref, w2t_ref, b2t_ref, out_ref, *,
                  inv_thw, inv_chw):
    f32 = jnp.float32
    hw = x_ref.shape[1]
    seg = _SEG if hw % _SEG == 0 else hw
    s_acc = None
    m_acc = None
    for k in range(hw // seg):
        blk = x_ref[0, k * seg:(k + 1) * seg]
        ps = jnp.sum(blk, axis=0)
        pm = jnp.max(blk, axis=0)
        s_acc = ps if s_acc is None else s_acc + ps
        m_acc = pm if m_acc is None else jnp.maximum(m_acc, pm)
    S, M = s_acc, m_acc

    c11 = (((1,), (1,)), ((), ()))
    a1 = jnp.sum(S, axis=0, keepdims=True) * inv_thw
    m1 = jnp.max(M, axis=0, keepdims=True)
    w1 = w1_ref[...]
    b1 = b1_ref[...]
    ha = jnp.maximum(
        jax.lax.dot_general(a1, w1, c11, preferred_element_type=f32) + b1, 0.0)
    hm = jnp.maximum(
        jax.lax.dot_general(m1, w1, c11, preferred_element_type=f32) + b1, 0.0)
    w2 = w2_ref[...]
    o1 = (jnp.dot(ha, w2, preferred_element_type=f32)
          + jnp.dot(hm, w2, preferred_element_type=f32)
          + 2.0 * b2_ref[...])
    scale = jax.nn.sigmoid(o1)

    pa = jnp.sum(S * scale, axis=1, keepdims=True).T * inv_chw
    pm2 = jnp.max(M * scale, axis=1, keepdims=True).T
    w1t = w1t_ref[...]
    b1t = b1t_ref[...]
    h2a = jnp.maximum(
        jax.lax.dot_general(pa, w1t, c11, preferred_element_type=f32) + b1t, 0.0)
    h2m = jnp.maximum(
        jax.lax.dot_general(pm2, w1t, c11, preferred_element_type=f32) + b1t, 0.0)
    w2t = w2t_ref[...]
    o2 = (jnp.dot(h2a, w2t, preferred_element_type=f32)
          + jnp.dot(h2m, w2t, preferred_element_type=f32)
          + 2.0 * b2t_ref[...])
    out_ref[...] = jax.nn.sigmoid(o2)[None]


def kernel(x, w1, b1, w2, b2, w1t, b1t, w2t, b2t):
    B, C, T, H, W = x.shape
    HW = H * W

    xt = jnp.transpose(x, (0, 3, 4, 2, 1)).reshape(B, HW, T, C)

    fused = functools.partial(
        _fused_kernel,
        inv_thw=1.0 / float(T * H * W), inv_chw=1.0 / float(C * H * W))
    zmap = lambda shape: (lambda b: tuple(0 for _ in shape))
    full = lambda a: pl.BlockSpec(a.shape, zmap(a.shape))
    w1v, b1t_r, w2t_r, b2t_r = w1.T, b1t.T, w2t.T, b2t.T
    mc2 = pl.pallas_call(
        fused,
        out_shape=jax.ShapeDtypeStruct((B, 1, T), jnp.float32),
        grid=(B,),
        in_specs=[pl.BlockSpec((1, HW, T, C), lambda b: (b, 0, 0, 0)),
                  full(w1v), full(b1), full(w2), full(b2),
                  full(w1t), full(b1t_r), full(w2t_r), full(b2t_r)],
        out_specs=pl.BlockSpec((1, 1, T), lambda b: (b, 0, 0)),
        compiler_params=pltpu.CompilerParams(
            dimension_semantics=("arbitrary",)),
    )(xt, w1v, b1, w2, b2, w1t, b1t_r, w2t_r, b2t_r)
    return mc2.reshape(B, T)
